# Initial kernel scaffold; baseline (speedup 1.0000x reference)
import jax
import jax.numpy as jnp
from jax.experimental import pallas as pl


def kernel(x, edge_index, batch, params):
    raise NotImplementedError("write your pallas kernel here")


# speedup vs baseline: 2.3118x; 2.3118x over previous
import functools

import jax
import jax.numpy as jnp
from jax.experimental import pallas as pl

HIDDEN = 128


def _mm_kernel(x_ref, w_ref, b_ref, o_ref, *, relu):
    o = jnp.dot(x_ref[...], w_ref[...], preferred_element_type=jnp.float32)
    o = o + b_ref[...]
    if relu:
        o = jnp.maximum(o, 0.0)
    o_ref[...] = o


def _dense(x, W, b, relu=False, bm=2048):
    N, K = x.shape
    M = W.shape[1]
    Np = ((N + bm - 1) // bm) * bm
    if Np != N:
        x = jnp.pad(x, ((0, Np - N), (0, 0)))
    out = pl.pallas_call(
        functools.partial(_mm_kernel, relu=relu),
        grid=(Np // bm,),
        in_specs=[
            pl.BlockSpec((bm, K), lambda i: (i, 0)),
            pl.BlockSpec((K, M), lambda i: (0, 0)),
            pl.BlockSpec((1, M), lambda i: (0, 0)),
        ],
        out_specs=pl.BlockSpec((bm, M), lambda i: (i, 0)),
        out_shape=jax.ShapeDtypeStruct((Np, M), jnp.float32),
    )(x, W, b.reshape(1, M))
    return out[:N]


def _gat_proj_kernel(x_ref, w_ref, asrc_ref, adst_ref, h_ref, s_ref, d_ref,
                     *, heads):
    h = jnp.dot(x_ref[...], w_ref[...], preferred_element_type=jnp.float32)
    h_ref[...] = h
    bm = h.shape[0]
    h3 = h.reshape(bm, heads, HIDDEN)
    s_ref[...] = (h3 * asrc_ref[...][None]).sum(-1)
    d_ref[...] = (h3 * adst_ref[...][None]).sum(-1)


def _gat_project(x, p, heads, bm=2048):
    N, K = x.shape
    M = p['W'].shape[1]
    Np = ((N + bm - 1) // bm) * bm
    if Np != N:
        x = jnp.pad(x, ((0, Np - N), (0, 0)))
    h, a_src, a_dst = pl.pallas_call(
        functools.partial(_gat_proj_kernel, heads=heads),
        grid=(Np // bm,),
        in_specs=[
            pl.BlockSpec((bm, K), lambda i: (i, 0)),
            pl.BlockSpec((K, M), lambda i: (0, 0)),
            pl.BlockSpec((heads, HIDDEN), lambda i: (0, 0)),
            pl.BlockSpec((heads, HIDDEN), lambda i: (0, 0)),
        ],
        out_specs=[
            pl.BlockSpec((bm, M), lambda i: (i, 0)),
            pl.BlockSpec((bm, heads), lambda i: (i, 0)),
            pl.BlockSpec((bm, heads), lambda i: (i, 0)),
        ],
        out_shape=[
            jax.ShapeDtypeStruct((Np, M), jnp.float32),
            jax.ShapeDtypeStruct((Np, heads), jnp.float32),
            jax.ShapeDtypeStruct((Np, heads), jnp.float32),
        ],
    )(x, p['W'], p['att_src'], p['att_dst'])
    return h[:N], a_src[:N], a_dst[:N]


def _edge_call(kern, outs_wide, be, *arrays):
    E = arrays[0].shape[0]
    Ep = ((E + be - 1) // be) * be
    padded = [jnp.pad(a, ((0, Ep - E), (0, 0))) if Ep != E else a for a in arrays]
    in_specs = [pl.BlockSpec((be, a.shape[1]), lambda i: (i, 0)) for a in padded]
    out_specs = [pl.BlockSpec((be, w), lambda i: (i, 0)) for w in outs_wide]
    out_shape = [jax.ShapeDtypeStruct((Ep, w), jnp.float32) for w in outs_wide]
    if len(outs_wide) == 1:
        out_specs, out_shape = out_specs[0], out_shape[0]
    res = pl.pallas_call(
        kern,
        grid=(Ep // be,),
        in_specs=in_specs,
        out_specs=out_specs,
        out_shape=out_shape,
    )(*padded)
    if len(outs_wide) == 1:
        return res[:E]
    return tuple(r[:E] for r in res)


def _ew_flat(kern, *arrays, bm=2048, lanes=512):
    shape = arrays[0].shape
    n = arrays[0].size
    chunk = bm * lanes
    npad = ((n + chunk - 1) // chunk) * chunk
    flat = [jnp.pad(a.reshape(-1), (0, npad - n)).reshape(npad // lanes, lanes)
            for a in arrays]
    R = npad // lanes
    res = pl.pallas_call(
        kern,
        grid=(R // bm,),
        in_specs=[pl.BlockSpec((bm, lanes), lambda i: (i, 0)) for _ in flat],
        out_specs=pl.BlockSpec((bm, lanes), lambda i: (i, 0)),
        out_shape=jax.ShapeDtypeStruct((R, lanes), jnp.float32),
    )(*flat)
    return res.reshape(-1)[:n].reshape(shape)


def _leaky_kernel(a_ref, b_ref, o_ref):
    e = a_ref[...] + b_ref[...]
    o_ref[...] = jnp.where(e >= 0.0, e, 0.2 * e)


def _expsub_kernel(e_ref, m_ref, o_ref):
    o_ref[...] = jnp.exp(e_ref[...] - m_ref[...])


def _div_kernel(p_ref, d_ref, o_ref):
    o_ref[...] = p_ref[...] / (d_ref[...] + 1e-16)


def _weight_msg_kernel(h_ref, a_ref, o_ref, *, heads):
    be, M = h_ref.shape
    h3 = h_ref[...].reshape(be, heads, HIDDEN)
    o_ref[...] = (h3 * a_ref[...][:, :, None]).reshape(be, M)


def _scale_kernel(h_ref, a_ref, o_ref):
    o_ref[...] = h_ref[...] * a_ref[...]


def _head_kernel(gat_ref, gcn_ref, wf1_ref, wf2_ref, bf_ref,
                 w1_ref, b1_ref, w2_ref, b2_ref, w3_ref, b3_ref,
                 wu1_ref, bu1_ref, wu2_ref, bu2_ref,
                 aff_ref, unc_ref):
    fused = jnp.dot(gat_ref[...], wf1_ref[...], preferred_element_type=jnp.float32)
    fused = fused + jnp.dot(gcn_ref[...], wf2_ref[...],
                            preferred_element_type=jnp.float32)
    fused = jnp.maximum(fused + bf_ref[...], 0.0)
    h = jnp.maximum(jnp.dot(fused, w1_ref[...],
                            preferred_element_type=jnp.float32) + b1_ref[...], 0.0)
    h = jnp.maximum(jnp.dot(h, w2_ref[...],
                            preferred_element_type=jnp.float32) + b2_ref[...], 0.0)
    aff_ref[...] = jnp.dot(h, w3_ref[...],
                           preferred_element_type=jnp.float32) + b3_ref[...]
    u = jnp.maximum(jnp.dot(fused, wu1_ref[...],
                            preferred_element_type=jnp.float32) + bu1_ref[...], 0.0)
    v = jnp.dot(u, wu2_ref[...], preferred_element_type=jnp.float32) + bu2_ref[...]
    vc = jnp.minimum(v, 30.0)
    unc_ref[...] = jnp.where(v > 30.0, v, jnp.log(1.0 + jnp.exp(vc)))


def _head(gat_pooled, gcn_pooled, params):
    G = gat_pooled.shape[0]
    fp = params['fusion']
    wf1 = fp['W'][:HIDDEN]
    wf2 = fp['W'][HIDDEN:]
    c0, c1, c2 = params['cls']
    u0, u1 = params['unc']
    args = [gat_pooled, gcn_pooled, wf1, wf2, fp['b'].reshape(1, -1),
            c0['W'], c0['b'].reshape(1, -1), c1['W'], c1['b'].reshape(1, -1),
            c2['W'], c2['b'].reshape(1, -1),
            u0['W'], u0['b'].reshape(1, -1), u1['W'], u1['b'].reshape(1, -1)]
    in_specs = [pl.BlockSpec(a.shape, lambda i: (0, 0)) for a in args]
    aff, unc = pl.pallas_call(
        _head_kernel,
        grid=(1,),
        in_specs=in_specs,
        out_specs=[pl.BlockSpec((G, 1), lambda i: (0, 0)),
                   pl.BlockSpec((G, 1), lambda i: (0, 0))],
        out_shape=[jax.ShapeDtypeStruct((G, 1), jnp.float32),
                   jax.ShapeDtypeStruct((G, 1), jnp.float32)],
    )(*args)
    return aff, unc


def kernel(x, edge_index, batch, params):
    N = x.shape[0]
    G = 64
    loop = jnp.arange(N, dtype=edge_index.dtype)
    src = jnp.concatenate([edge_index[0], loop])
    dst = jnp.concatenate([edge_index[1], loop])

    heads_list = [4, 4, 1]
    gat_x = x
    for i, p in enumerate(params['gat']):
        heads = heads_list[i]
        h, a_src, a_dst = _gat_project(gat_x, p, heads)
        e = _ew_flat(_leaky_kernel, a_src[src], a_dst[dst])
        m = jax.ops.segment_max(e, dst, num_segments=N)
        pexp = _ew_flat(_expsub_kernel, e, m[dst])
        denom = jax.ops.segment_sum(pexp, dst, num_segments=N)
        alpha = _ew_flat(_div_kernel, pexp, denom[dst])
        msg = _edge_call(functools.partial(_weight_msg_kernel, heads=heads),
                         [heads * HIDDEN], 4096, h[src], alpha)
        gat_x = jax.ops.segment_sum(msg, dst, num_segments=N) + p['b']
        if i < len(params['gat']) - 1:
            gat_x = jnp.maximum(gat_x, 0.0)

    ones = jnp.ones((dst.shape[0],), jnp.float32)
    deg = jax.ops.segment_sum(ones, dst, num_segments=N)
    dinv = jnp.where(deg > 0, jax.lax.rsqrt(jnp.maximum(deg, 1e-12)), 0.0)
    dinv = dinv.reshape(N, 1)
    norm = (dinv[src] * dinv[dst])

    gcn_x = x
    for p in params['gcn']:
        h = _dense(gcn_x, p['W'], jnp.zeros_like(p['b']))
        msg = _edge_call(_scale_kernel, [h.shape[1]], 8192, h[src], norm)
        agg = jax.ops.segment_sum(msg, dst, num_segments=N) + p['b']
        gcn_x = jnp.maximum(agg, 0.0)

    cnt = jax.ops.segment_sum(jnp.ones((N,), jnp.float32), batch, num_segments=G)
    cnt = jnp.maximum(cnt, 1.0).reshape(G, 1)
    gat_pooled = jax.ops.segment_sum(gat_x, batch, num_segments=G) / cnt
    gcn_pooled = jax.ops.segment_sum(gcn_x, batch, num_segments=G) / cnt

    return _head(gat_pooled, gcn_pooled, params)


# speedup vs baseline: 3.2371x; 1.4003x over previous
import functools

import jax
import jax.numpy as jnp
from jax.experimental import pallas as pl

HIDDEN = 128


def _mm_kernel(x_ref, w_ref, b_ref, o_ref, *, relu):
    o = jnp.dot(x_ref[...], w_ref[...], preferred_element_type=jnp.float32)
    o = o + b_ref[...]
    if relu:
        o = jnp.maximum(o, 0.0)
    o_ref[...] = o


def _dense(x, W, b, relu=False, bm=2048):
    N, K = x.shape
    M = W.shape[1]
    Np = ((N + bm - 1) // bm) * bm
    if Np != N:
        x = jnp.pad(x, ((0, Np - N), (0, 0)))
    out = pl.pallas_call(
        functools.partial(_mm_kernel, relu=relu),
        grid=(Np // bm,),
        in_specs=[
            pl.BlockSpec((bm, K), lambda i: (i, 0)),
            pl.BlockSpec((K, M), lambda i: (0, 0)),
            pl.BlockSpec((1, M), lambda i: (0, 0)),
        ],
        out_specs=pl.BlockSpec((bm, M), lambda i: (i, 0)),
        out_shape=jax.ShapeDtypeStruct((Np, M), jnp.float32),
    )(x, W, b.reshape(1, M))
    return out[:N]


def _gat_proj_kernel(x_ref, w_ref, asrc_ref, adst_ref, h_ref, s_ref, d_ref,
                     *, heads):
    h = jnp.dot(x_ref[...], w_ref[...], preferred_element_type=jnp.float32)
    h_ref[...] = h
    bm = h.shape[0]
    h3 = h.reshape(bm, heads, HIDDEN)
    s_ref[...] = (h3 * asrc_ref[...][None]).sum(-1)
    d_ref[...] = (h3 * adst_ref[...][None]).sum(-1)


def _gat_project(x, p, heads, bm=2048):
    N, K = x.shape
    M = p['W'].shape[1]
    Np = ((N + bm - 1) // bm) * bm
    if Np != N:
        x = jnp.pad(x, ((0, Np - N), (0, 0)))
    h, a_src, a_dst = pl.pallas_call(
        functools.partial(_gat_proj_kernel, heads=heads),
        grid=(Np // bm,),
        in_specs=[
            pl.BlockSpec((bm, K), lambda i: (i, 0)),
            pl.BlockSpec((K, M), lambda i: (0, 0)),
            pl.BlockSpec((heads, HIDDEN), lambda i: (0, 0)),
            pl.BlockSpec((heads, HIDDEN), lambda i: (0, 0)),
        ],
        out_specs=[
            pl.BlockSpec((bm, M), lambda i: (i, 0)),
            pl.BlockSpec((bm, heads), lambda i: (i, 0)),
            pl.BlockSpec((bm, heads), lambda i: (i, 0)),
        ],
        out_shape=[
            jax.ShapeDtypeStruct((Np, M), jnp.float32),
            jax.ShapeDtypeStruct((Np, heads), jnp.float32),
            jax.ShapeDtypeStruct((Np, heads), jnp.float32),
        ],
    )(x, p['W'], p['att_src'], p['att_dst'])
    return h[:N], a_src[:N], a_dst[:N]


def _edge_call(kern, outs_wide, be, *arrays):
    E = arrays[0].shape[0]
    Ep = ((E + be - 1) // be) * be
    padded = [jnp.pad(a, ((0, Ep - E), (0, 0))) if Ep != E else a for a in arrays]
    in_specs = [pl.BlockSpec((be, a.shape[1]), lambda i: (i, 0)) for a in padded]
    out_specs = [pl.BlockSpec((be, w), lambda i: (i, 0)) for w in outs_wide]
    out_shape = [jax.ShapeDtypeStruct((Ep, w), jnp.float32) for w in outs_wide]
    if len(outs_wide) == 1:
        out_specs, out_shape = out_specs[0], out_shape[0]
    res = pl.pallas_call(
        kern,
        grid=(Ep // be,),
        in_specs=in_specs,
        out_specs=out_specs,
        out_shape=out_shape,
    )(*padded)
    if len(outs_wide) == 1:
        return res[:E]
    return tuple(r[:E] for r in res)


def _ew_flat(kern, *arrays, bm=2048, lanes=512):
    shape = arrays[0].shape
    n = arrays[0].size
    chunk = bm * lanes
    npad = ((n + chunk - 1) // chunk) * chunk
    flat = [jnp.pad(a.reshape(-1), (0, npad - n)).reshape(npad // lanes, lanes)
            for a in arrays]
    R = npad // lanes
    res = pl.pallas_call(
        kern,
        grid=(R // bm,),
        in_specs=[pl.BlockSpec((bm, lanes), lambda i: (i, 0)) for _ in flat],
        out_specs=pl.BlockSpec((bm, lanes), lambda i: (i, 0)),
        out_shape=jax.ShapeDtypeStruct((R, lanes), jnp.float32),
    )(*flat)
    return res.reshape(-1)[:n].reshape(shape)


def _leakyexp_kernel(a_ref, b_ref, o_ref):
    e = a_ref[...] + b_ref[...]
    e = jnp.where(e >= 0.0, e, 0.2 * e)
    o_ref[...] = jnp.exp(e)


def _div_kernel(p_ref, d_ref, o_ref):
    o_ref[...] = p_ref[...] / (d_ref[...] + 1e-16)


def _weight_msg_kernel(h_ref, a_ref, o_ref, *, heads):
    be, M = h_ref.shape
    h3 = h_ref[...].reshape(be, heads, HIDDEN)
    o_ref[...] = (h3 * a_ref[...][:, :, None]).reshape(be, M)


def _mm_scale_kernel(x_ref, w_ref, s_ref, o_ref):
    o_ref[...] = jnp.dot(x_ref[...], w_ref[...],
                         preferred_element_type=jnp.float32) * s_ref[...]


def _dense_prescaled(x, W, s, bm=2048):
    N, K = x.shape
    M = W.shape[1]
    Np = ((N + bm - 1) // bm) * bm
    if Np != N:
        x = jnp.pad(x, ((0, Np - N), (0, 0)))
        s = jnp.pad(s, ((0, Np - N), (0, 0)))
    out = pl.pallas_call(
        _mm_scale_kernel,
        grid=(Np // bm,),
        in_specs=[
            pl.BlockSpec((bm, K), lambda i: (i, 0)),
            pl.BlockSpec((K, M), lambda i: (0, 0)),
            pl.BlockSpec((bm, 1), lambda i: (i, 0)),
        ],
        out_specs=pl.BlockSpec((bm, M), lambda i: (i, 0)),
        out_shape=jax.ShapeDtypeStruct((Np, M), jnp.float32),
    )(x, W, s)
    return out[:N]


def _postscale_kernel(a_ref, s_ref, b_ref, o_ref):
    o_ref[...] = jnp.maximum(a_ref[...] * s_ref[...] + b_ref[...], 0.0)


def _postscale(agg, s, b, bm=2048):
    N, M = agg.shape
    Np = ((N + bm - 1) // bm) * bm
    if Np != N:
        agg = jnp.pad(agg, ((0, Np - N), (0, 0)))
        s = jnp.pad(s, ((0, Np - N), (0, 0)))
    out = pl.pallas_call(
        _postscale_kernel,
        grid=(Np // bm,),
        in_specs=[
            pl.BlockSpec((bm, M), lambda i: (i, 0)),
            pl.BlockSpec((bm, 1), lambda i: (i, 0)),
            pl.BlockSpec((1, M), lambda i: (0, 0)),
        ],
        out_specs=pl.BlockSpec((bm, M), lambda i: (i, 0)),
        out_shape=jax.ShapeDtypeStruct((Np, M), jnp.float32),
    )(agg, s, b.reshape(1, M))
    return out[:N]


def _head_kernel(gat_ref, gcn_ref, wf1_ref, wf2_ref, bf_ref,
                 w1_ref, b1_ref, w2_ref, b2_ref, w3_ref, b3_ref,
                 wu1_ref, bu1_ref, wu2_ref, bu2_ref,
                 aff_ref, unc_ref):
    fused = jnp.dot(gat_ref[...], wf1_ref[...], preferred_element_type=jnp.float32)
    fused = fused + jnp.dot(gcn_ref[...], wf2_ref[...],
                            preferred_element_type=jnp.float32)
    fused = jnp.maximum(fused + bf_ref[...], 0.0)
    h = jnp.maximum(jnp.dot(fused, w1_ref[...],
                            preferred_element_type=jnp.float32) + b1_ref[...], 0.0)
    h = jnp.maximum(jnp.dot(h, w2_ref[...],
                            preferred_element_type=jnp.float32) + b2_ref[...], 0.0)
    aff_ref[...] = jnp.dot(h, w3_ref[...],
                           preferred_element_type=jnp.float32) + b3_ref[...]
    u = jnp.maximum(jnp.dot(fused, wu1_ref[...],
                            preferred_element_type=jnp.float32) + bu1_ref[...], 0.0)
    v = jnp.dot(u, wu2_ref[...], preferred_element_type=jnp.float32) + bu2_ref[...]
    vc = jnp.minimum(v, 30.0)
    unc_ref[...] = jnp.where(v > 30.0, v, jnp.log(1.0 + jnp.exp(vc)))


def _head(gat_pooled, gcn_pooled, params):
    G = gat_pooled.shape[0]
    fp = params['fusion']
    wf1 = fp['W'][:HIDDEN]
    wf2 = fp['W'][HIDDEN:]
    c0, c1, c2 = params['cls']
    u0, u1 = params['unc']
    args = [gat_pooled, gcn_pooled, wf1, wf2, fp['b'].reshape(1, -1),
            c0['W'], c0['b'].reshape(1, -1), c1['W'], c1['b'].reshape(1, -1),
            c2['W'], c2['b'].reshape(1, -1),
            u0['W'], u0['b'].reshape(1, -1), u1['W'], u1['b'].reshape(1, -1)]
    in_specs = [pl.BlockSpec(a.shape, lambda i: (0, 0)) for a in args]
    aff, unc = pl.pallas_call(
        _head_kernel,
        grid=(1,),
        in_specs=in_specs,
        out_specs=[pl.BlockSpec((G, 1), lambda i: (0, 0)),
                   pl.BlockSpec((G, 1), lambda i: (0, 0))],
        out_shape=[jax.ShapeDtypeStruct((G, 1), jnp.float32),
                   jax.ShapeDtypeStruct((G, 1), jnp.float32)],
    )(*args)
    return aff, unc


def kernel(x, edge_index, batch, params):
    N = x.shape[0]
    G = 64
    loop = jnp.arange(N, dtype=edge_index.dtype)
    src = jnp.concatenate([edge_index[0], loop])
    dst = jnp.concatenate([edge_index[1], loop])

    heads_list = [4, 4, 1]
    gat_x = x
    for i, p in enumerate(params['gat']):
        heads = heads_list[i]
        h, a_src, a_dst = _gat_project(gat_x, p, heads)
        pexp = _ew_flat(_leakyexp_kernel, a_src[src], a_dst[dst])
        denom = jax.ops.segment_sum(pexp, dst, num_segments=N)
        alpha = _ew_flat(_div_kernel, pexp, denom[dst])
        msg = _edge_call(functools.partial(_weight_msg_kernel, heads=heads),
                         [heads * HIDDEN], 4096, h[src], alpha)
        gat_x = jax.ops.segment_sum(msg, dst, num_segments=N) + p['b']
        if i < len(params['gat']) - 1:
            gat_x = jnp.maximum(gat_x, 0.0)

    ones = jnp.ones((dst.shape[0],), jnp.float32)
    deg = jax.ops.segment_sum(ones, dst, num_segments=N)
    dinv = jnp.where(deg > 0, jax.lax.rsqrt(jnp.maximum(deg, 1e-12)), 0.0)
    dinv = dinv.reshape(N, 1)

    gcn_x = x
    for p in params['gcn']:
        hs = _dense_prescaled(gcn_x, p['W'], dinv)
        agg = jax.ops.segment_sum(hs[src], dst, num_segments=N)
        gcn_x = _postscale(agg, dinv, p['b'])

    cnt = jax.ops.segment_sum(jnp.ones((N,), jnp.float32), batch, num_segments=G)
    cnt = jnp.maximum(cnt, 1.0).reshape(G, 1)
    gat_pooled = jax.ops.segment_sum(gat_x, batch, num_segments=G) / cnt
    gcn_pooled = jax.ops.segment_sum(gcn_x, batch, num_segments=G) / cnt

    return _head(gat_pooled, gcn_pooled, params)


# speedup vs baseline: 3.7119x; 1.1467x over previous
import functools

import jax
import jax.numpy as jnp
from jax.experimental import pallas as pl

HIDDEN = 128


def _mm_kernel(x_ref, w_ref, b_ref, o_ref, *, relu):
    o = jnp.dot(x_ref[...], w_ref[...], preferred_element_type=jnp.float32)
    o = o + b_ref[...]
    if relu:
        o = jnp.maximum(o, 0.0)
    o_ref[...] = o


def _dense(x, W, b, relu=False, bm=2048):
    N, K = x.shape
    M = W.shape[1]
    Np = ((N + bm - 1) // bm) * bm
    if Np != N:
        x = jnp.pad(x, ((0, Np - N), (0, 0)))
    out = pl.pallas_call(
        functools.partial(_mm_kernel, relu=relu),
        grid=(Np // bm,),
        in_specs=[
            pl.BlockSpec((bm, K), lambda i: (i, 0)),
            pl.BlockSpec((K, M), lambda i: (0, 0)),
            pl.BlockSpec((1, M), lambda i: (0, 0)),
        ],
        out_specs=pl.BlockSpec((bm, M), lambda i: (i, 0)),
        out_shape=jax.ShapeDtypeStruct((Np, M), jnp.float32),
    )(x, W, b.reshape(1, M))
    return out[:N]


def _gat_proj_kernel(x_ref, w_ref, asrc_ref, adst_ref, h_ref, s_ref, d_ref,
                     *, heads):
    h = jnp.dot(x_ref[...], w_ref[...], preferred_element_type=jnp.float32)
    h_ref[...] = h
    bm = h.shape[0]
    h3 = h.reshape(bm, heads, HIDDEN)
    s_ref[...] = (h3 * asrc_ref[...][None]).sum(-1)
    d_ref[...] = (h3 * adst_ref[...][None]).sum(-1)


def _gat_project(x, p, heads, bm=2048):
    N, K = x.shape
    M = p['W'].shape[1]
    Np = ((N + bm - 1) // bm) * bm
    if Np != N:
        x = jnp.pad(x, ((0, Np - N), (0, 0)))
    h, a_src, a_dst = pl.pallas_call(
        functools.partial(_gat_proj_kernel, heads=heads),
        grid=(Np // bm,),
        in_specs=[
            pl.BlockSpec((bm, K), lambda i: (i, 0)),
            pl.BlockSpec((K, M), lambda i: (0, 0)),
            pl.BlockSpec((heads, HIDDEN), lambda i: (0, 0)),
            pl.BlockSpec((heads, HIDDEN), lambda i: (0, 0)),
        ],
        out_specs=[
            pl.BlockSpec((bm, M), lambda i: (i, 0)),
            pl.BlockSpec((bm, heads), lambda i: (i, 0)),
            pl.BlockSpec((bm, heads), lambda i: (i, 0)),
        ],
        out_shape=[
            jax.ShapeDtypeStruct((Np, M), jnp.float32),
            jax.ShapeDtypeStruct((Np, heads), jnp.float32),
            jax.ShapeDtypeStruct((Np, heads), jnp.float32),
        ],
    )(x, p['W'], p['att_src'], p['att_dst'])
    return h[:N], a_src[:N], a_dst[:N]


def _edge_call(kern, outs_wide, be, *arrays):
    E = arrays[0].shape[0]
    Ep = ((E + be - 1) // be) * be
    padded = [jnp.pad(a, ((0, Ep - E), (0, 0))) if Ep != E else a for a in arrays]
    in_specs = [pl.BlockSpec((be, a.shape[1]), lambda i: (i, 0)) for a in padded]
    out_specs = [pl.BlockSpec((be, w), lambda i: (i, 0)) for w in outs_wide]
    out_shape = [jax.ShapeDtypeStruct((Ep, w), jnp.float32) for w in outs_wide]
    if len(outs_wide) == 1:
        out_specs, out_shape = out_specs[0], out_shape[0]
    res = pl.pallas_call(
        kern,
        grid=(Ep // be,),
        in_specs=in_specs,
        out_specs=out_specs,
        out_shape=out_shape,
    )(*padded)
    if len(outs_wide) == 1:
        return res[:E]
    return tuple(r[:E] for r in res)


def _ew_flat(kern, *arrays, bm=2048, lanes=512):
    shape = arrays[0].shape
    n = arrays[0].size
    chunk = bm * lanes
    npad = ((n + chunk - 1) // chunk) * chunk
    flat = [jnp.pad(a.reshape(-1), (0, npad - n)).reshape(npad // lanes, lanes)
            for a in arrays]
    R = npad // lanes
    res = pl.pallas_call(
        kern,
        grid=(R // bm,),
        in_specs=[pl.BlockSpec((bm, lanes), lambda i: (i, 0)) for _ in flat],
        out_specs=pl.BlockSpec((bm, lanes), lambda i: (i, 0)),
        out_shape=jax.ShapeDtypeStruct((R, lanes), jnp.float32),
    )(*flat)
    return res.reshape(-1)[:n].reshape(shape)


def _leakyexp_kernel(a_ref, b_ref, o_ref):
    e = a_ref[...] + b_ref[...]
    e = jnp.where(e >= 0.0, e, 0.2 * e)
    o_ref[...] = jnp.exp(e)


def _gat_out_kernel(a_ref, d_ref, b_ref, o_ref, *, heads, relu):
    bm, M = a_ref.shape
    a3 = a_ref[...].reshape(bm, heads, HIDDEN)
    o = (a3 / (d_ref[...][:, :, None] + 1e-16)).reshape(bm, M) + b_ref[...]
    if relu:
        o = jnp.maximum(o, 0.0)
    o_ref[...] = o


def _gat_out(agg, denom, b, heads, relu, bm=2048):
    N, M = agg.shape
    Np = ((N + bm - 1) // bm) * bm
    if Np != N:
        agg = jnp.pad(agg, ((0, Np - N), (0, 0)))
        denom = jnp.pad(denom, ((0, Np - N), (0, 0)))
    out = pl.pallas_call(
        functools.partial(_gat_out_kernel, heads=heads, relu=relu),
        grid=(Np // bm,),
        in_specs=[
            pl.BlockSpec((bm, M), lambda i: (i, 0)),
            pl.BlockSpec((bm, heads), lambda i: (i, 0)),
            pl.BlockSpec((1, M), lambda i: (0, 0)),
        ],
        out_specs=pl.BlockSpec((bm, M), lambda i: (i, 0)),
        out_shape=jax.ShapeDtypeStruct((Np, M), jnp.float32),
    )(agg, denom, b.reshape(1, M))
    return out[:N]


def _weight_msg_kernel(h_ref, a_ref, o_ref, *, heads):
    be, M = h_ref.shape
    h3 = h_ref[...].reshape(be, heads, HIDDEN)
    o_ref[...] = (h3 * a_ref[...][:, :, None]).reshape(be, M)


def _mm_scale_kernel(x_ref, w_ref, s_ref, o_ref):
    o_ref[...] = jnp.dot(x_ref[...], w_ref[...],
                         preferred_element_type=jnp.float32) * s_ref[...]


def _dense_prescaled(x, W, s, bm=2048):
    N, K = x.shape
    M = W.shape[1]
    Np = ((N + bm - 1) // bm) * bm
    if Np != N:
        x = jnp.pad(x, ((0, Np - N), (0, 0)))
        s = jnp.pad(s, ((0, Np - N), (0, 0)))
    out = pl.pallas_call(
        _mm_scale_kernel,
        grid=(Np // bm,),
        in_specs=[
            pl.BlockSpec((bm, K), lambda i: (i, 0)),
            pl.BlockSpec((K, M), lambda i: (0, 0)),
            pl.BlockSpec((bm, 1), lambda i: (i, 0)),
        ],
        out_specs=pl.BlockSpec((bm, M), lambda i: (i, 0)),
        out_shape=jax.ShapeDtypeStruct((Np, M), jnp.float32),
    )(x, W, s)
    return out[:N]


def _postscale_kernel(a_ref, s_ref, b_ref, o_ref):
    o_ref[...] = jnp.maximum(a_ref[...] * s_ref[...] + b_ref[...], 0.0)


def _postscale(agg, s, b, bm=2048):
    N, M = agg.shape
    Np = ((N + bm - 1) // bm) * bm
    if Np != N:
        agg = jnp.pad(agg, ((0, Np - N), (0, 0)))
        s = jnp.pad(s, ((0, Np - N), (0, 0)))
    out = pl.pallas_call(
        _postscale_kernel,
        grid=(Np // bm,),
        in_specs=[
            pl.BlockSpec((bm, M), lambda i: (i, 0)),
            pl.BlockSpec((bm, 1), lambda i: (i, 0)),
            pl.BlockSpec((1, M), lambda i: (0, 0)),
        ],
        out_specs=pl.BlockSpec((bm, M), lambda i: (i, 0)),
        out_shape=jax.ShapeDtypeStruct((Np, M), jnp.float32),
    )(agg, s, b.reshape(1, M))
    return out[:N]


def _head_kernel(gat_ref, gcn_ref, wf1_ref, wf2_ref, bf_ref,
                 w1_ref, b1_ref, w2_ref, b2_ref, w3_ref, b3_ref,
                 wu1_ref, bu1_ref, wu2_ref, bu2_ref,
                 aff_ref, unc_ref):
    fused = jnp.dot(gat_ref[...], wf1_ref[...], preferred_element_type=jnp.float32)
    fused = fused + jnp.dot(gcn_ref[...], wf2_ref[...],
                            preferred_element_type=jnp.float32)
    fused = jnp.maximum(fused + bf_ref[...], 0.0)
    h = jnp.maximum(jnp.dot(fused, w1_ref[...],
                            preferred_element_type=jnp.float32) + b1_ref[...], 0.0)
    h = jnp.maximum(jnp.dot(h, w2_ref[...],
                            preferred_element_type=jnp.float32) + b2_ref[...], 0.0)
    aff_ref[...] = jnp.dot(h, w3_ref[...],
                           preferred_element_type=jnp.float32) + b3_ref[...]
    u = jnp.maximum(jnp.dot(fused, wu1_ref[...],
                            preferred_element_type=jnp.float32) + bu1_ref[...], 0.0)
    v = jnp.dot(u, wu2_ref[...], preferred_element_type=jnp.float32) + bu2_ref[...]
    vc = jnp.minimum(v, 30.0)
    unc_ref[...] = jnp.where(v > 30.0, v, jnp.log(1.0 + jnp.exp(vc)))


def _head(gat_pooled, gcn_pooled, params):
    G = gat_pooled.shape[0]
    fp = params['fusion']
    wf1 = fp['W'][:HIDDEN]
    wf2 = fp['W'][HIDDEN:]
    c0, c1, c2 = params['cls']
    u0, u1 = params['unc']
    args = [gat_pooled, gcn_pooled, wf1, wf2, fp['b'].reshape(1, -1),
            c0['W'], c0['b'].reshape(1, -1), c1['W'], c1['b'].reshape(1, -1),
            c2['W'], c2['b'].reshape(1, -1),
            u0['W'], u0['b'].reshape(1, -1), u1['W'], u1['b'].reshape(1, -1)]
    in_specs = [pl.BlockSpec(a.shape, lambda i: (0, 0)) for a in args]
    aff, unc = pl.pallas_call(
        _head_kernel,
        grid=(1,),
        in_specs=in_specs,
        out_specs=[pl.BlockSpec((G, 1), lambda i: (0, 0)),
                   pl.BlockSpec((G, 1), lambda i: (0, 0))],
        out_shape=[jax.ShapeDtypeStruct((G, 1), jnp.float32),
                   jax.ShapeDtypeStruct((G, 1), jnp.float32)],
    )(*args)
    return aff, unc


def kernel(x, edge_index, batch, params):
    N = x.shape[0]
    G = 64
    loop = jnp.arange(N, dtype=edge_index.dtype)
    src = jnp.concatenate([edge_index[0], loop])
    dst = jnp.concatenate([edge_index[1], loop])

    heads_list = [4, 4, 1]
    gat_x = x
    for i, p in enumerate(params['gat']):
        heads = heads_list[i]
        h, a_src, a_dst = _gat_project(gat_x, p, heads)
        pexp = _ew_flat(_leakyexp_kernel, a_src[src], a_dst[dst])
        denom = jax.ops.segment_sum(pexp, dst, num_segments=N)
        msg = _edge_call(functools.partial(_weight_msg_kernel, heads=heads),
                         [heads * HIDDEN], 4096, h[src], pexp)
        agg = jax.ops.segment_sum(msg, dst, num_segments=N)
        gat_x = _gat_out(agg, denom, p['b'], heads,
                         relu=i < len(params['gat']) - 1)

    ones = jnp.ones((dst.shape[0],), jnp.float32)
    deg = jax.ops.segment_sum(ones, dst, num_segments=N)
    dinv = jnp.where(deg > 0, jax.lax.rsqrt(jnp.maximum(deg, 1e-12)), 0.0)
    dinv = dinv.reshape(N, 1)

    gcn_x = x
    for p in params['gcn']:
        hs = _dense_prescaled(gcn_x, p['W'], dinv)
        agg = jax.ops.segment_sum(hs[src], dst, num_segments=N)
        gcn_x = _postscale(agg, dinv, p['b'])

    cnt = jax.ops.segment_sum(jnp.ones((N,), jnp.float32), batch, num_segments=G)
    cnt = jnp.maximum(cnt, 1.0).reshape(G, 1)
    gat_pooled = jax.ops.segment_sum(gat_x, batch, num_segments=G) / cnt
    gcn_pooled = jax.ops.segment_sum(gcn_x, batch, num_segments=G) / cnt

    return _head(gat_pooled, gcn_pooled, params)
